# Initial kernel scaffold; baseline (speedup 1.0000x reference)
import jax
import jax.numpy as jnp
from jax.experimental import pallas as pl


def kernel(x, edge_index, edge_attr, params):
    raise NotImplementedError("write your pallas kernel here")


# speedup vs baseline: 7.6638x; 7.6638x over previous
import functools

import jax
import jax.numpy as jnp
from jax import lax
from jax.experimental import pallas as pl
from jax.experimental.pallas import tpu as pltpu
from jax.experimental.pallas import tpu_sc as plsc

N = 10000
E = 320000
D_IN = 128
H = 64
HEADS = 8
CH = H // HEADS

NW = 32
NPT = 313
NPAD = NW * NPT
C = 128
EPW = 79 * C
EPAD = EPW * NW
BE = 2048
BN = 2504
NPS = NPAD // 16

@functools.lru_cache(maxsize=None)
def _mesh():
    return plsc.VectorSubcoreMesh(core_axis_name="c", subcore_axis_name="s")


@functools.lru_cache(maxsize=None)
def _sc_gather(D):
    @functools.partial(
        pl.kernel,
        out_type=jax.ShapeDtypeStruct((EPAD, D), jnp.float32),
        mesh=_mesh(),
        scratch_types=[
            pltpu.VMEM((C,), jnp.int32),
            pltpu.VMEM((C, D), jnp.float32),
            pltpu.SemaphoreType.DMA,
        ],
        name=f"sc_gather_{D}",
        compiler_params=pltpu.CompilerParams(use_tc_tiling_on_sc=False),
    )
    def k(table, idx, out, idx_v, rows_v, sem):
        w = lax.axis_index("s") * 2 + lax.axis_index("c")
        base = w * EPW

        def body(i, carry):
            off = base + i * C
            pltpu.sync_copy(idx.at[pl.ds(off, C)], idx_v)
            pltpu.async_copy(table.at[idx_v], rows_v, sem).wait()
            pltpu.sync_copy(rows_v, out.at[pl.ds(off, C)])
            return carry

        lax.fori_loop(0, EPW // C, body, 0)

    return k


@functools.lru_cache(maxsize=None)
def _sc_scatter_add(D):
    @functools.partial(
        pl.kernel,
        out_type=jax.ShapeDtypeStruct((2, NPAD, D), jnp.float32),
        mesh=_mesh(),
        scratch_types=[
            pltpu.VMEM((C,), jnp.int32),
            pltpu.VMEM((C, D), jnp.float32),
            pltpu.VMEM((NPS, D), jnp.float32),
            pltpu.VMEM_SHARED((NPAD, D), jnp.float32),
        ],
        name=f"sc_scatter_add_{D}",
        compiler_params=pltpu.CompilerParams(use_tc_tiling_on_sc=False),
    )
    def k(vals, idx, zeros, out, idx_v, vals_v, zrow_v, acc):
        c = lax.axis_index("c")
        s = lax.axis_index("s")
        w = s * 2 + c
        pltpu.sync_copy(zeros, zrow_v)
        pltpu.sync_copy(zrow_v, acc.at[pl.ds(s * NPS, NPS)])
        plsc.subcore_barrier()

        base = w * EPW

        def body(i, carry):
            off = base + i * C
            pltpu.sync_copy(idx.at[pl.ds(off, C)], idx_v)
            pltpu.sync_copy(vals.at[pl.ds(off, C)], vals_v)
            pltpu.sync_copy(vals_v, acc.at[idx_v], add=True)
            return carry

        lax.fori_loop(0, EPW // C, body, 0)
        plsc.subcore_barrier()
        pltpu.sync_copy(acc.at[pl.ds(s * NPS, NPS)],
                        out.at[c].at[pl.ds(s * NPS, NPS)])

    return k


@functools.lru_cache(maxsize=None)
def _sc_segmax():
    NEG = jnp.float32(-jnp.inf)

    @functools.partial(
        pl.kernel,
        out_type=jax.ShapeDtypeStruct((NPAD * H,), jnp.float32),
        mesh=_mesh(),
        scratch_types=[
            pltpu.VMEM((48,), jnp.int32),
            pltpu.VMEM((C,), jnp.int32),
            pltpu.VMEM((C * H,), jnp.float32),
            pltpu.VMEM((NPT * H,), jnp.float32),
        ],
        name="sc_segmax",
        compiler_params=pltpu.CompilerParams(use_tc_tiling_on_sc=False),
    )
    def k(m_flat, dsts, bnd, out, bnd_v, dst_v, vals_v, acc):
        w = lax.axis_index("s") * 2 + lax.axis_index("c")
        base = w * NPT

        def init(i, carry):
            acc[pl.ds(i * 16, 16)] = jnp.full((16,), NEG, jnp.float32)
            return carry

        lax.fori_loop(0, NPT * H // 16, init, 0)

        pltpu.sync_copy(bnd, bnd_v)
        bv = bnd_v[pl.ds(w, 16)]
        start = bv[0]
        end = bv[1]
        c0 = (start // C) * C
        nch = (end - c0 + C - 1) // C
        nch = jnp.maximum(nch, 0)

        def chunk(kk, carry):
            off = c0 + kk * C
            pltpu.sync_copy(m_flat.at[pl.ds(off * H, C * H)], vals_v)
            pltpu.sync_copy(dsts.at[pl.ds(off, C)], dst_v)

            def grp(gg, cc2):
                dvec = dst_v[pl.ds(gg * 16, 16)] - base
                for lane in range(16):
                    e = off + gg * 16 + lane
                    valid = jnp.logical_and(e >= start, e < end)
                    dloc = jnp.where(valid, dvec[lane], 0)
                    abase = dloc * H
                    vbase = (gg * 16 + lane) * H
                    for cc in range(H // 16):
                        v = vals_v[pl.ds(vbase + cc * 16, 16)]
                        v = jnp.where(valid, v,
                                      jnp.full((16,), NEG, jnp.float32))
                        a = acc[pl.ds(abase + cc * 16, 16)]
                        acc[pl.ds(abase + cc * 16, 16)] = jnp.maximum(a, v)
                return cc2

            lax.fori_loop(0, C // 16, grp, 0)
            return carry

        lax.fori_loop(0, nch, chunk, 0)
        pltpu.sync_copy(acc, out.at[pl.ds(base * H, NPT * H)])

    return k


def _dot(a, b):
    return jnp.dot(a, b, preferred_element_type=jnp.float32)


def _full(shape):
    return pl.BlockSpec(shape, lambda *_: (0,) * len(shape))


def _ln(z, g, b, eps=1e-5):
    mu = jnp.mean(z, axis=-1, keepdims=True)
    va = jnp.mean((z - mu) * (z - mu), axis=-1, keepdims=True)
    return (z - mu) / jnp.sqrt(va + eps) * g + b


def _expand_mat():
    hh = lax.broadcasted_iota(jnp.int32, (HEADS, H), 0)
    jj = lax.broadcasted_iota(jnp.int32, (HEADS, H), 1)
    return (jj // CH == hh).astype(jnp.float32)


def _tc_enc(x_p, W, b, g, bb):
    def body(x_ref, W_ref, b_ref, g_ref, bb_ref, o_ref):
        t = _dot(x_ref[...], W_ref[...]) + b_ref[...]
        t = t / jnp.sqrt(jnp.float32(1.0 + 1e-5)) * g_ref[...] + bb_ref[...]
        o_ref[...] = jnp.maximum(t, 0.0)

    return pl.pallas_call(
        body,
        grid=(NPAD // BN,),
        in_specs=[pl.BlockSpec((BN, D_IN), lambda i: (i, 0)),
                  _full((D_IN, H)), _full((1, H)), _full((1, H)), _full((1, H))],
        out_specs=pl.BlockSpec((BN, H), lambda i: (i, 0)),
        out_shape=jax.ShapeDtypeStruct((NPAD, H), jnp.float32),
    )(x_p, W, b.reshape(1, H), g.reshape(1, H), bb.reshape(1, H))


def _tc_eh(ea_p, W, b, g, bb):
    def body(a_ref, W_ref, b_ref, g_ref, bb_ref, o_ref):
        t = jnp.maximum(_dot(a_ref[...], W_ref[...]) + b_ref[...], 0.0)
        o_ref[...] = _ln(t, g_ref[...], bb_ref[...])

    return pl.pallas_call(
        body,
        grid=(EPAD // BE,),
        in_specs=[pl.BlockSpec((BE, 16), lambda i: (i, 0)),
                  _full((16, H)), _full((1, H)), _full((1, H)), _full((1, H))],
        out_specs=pl.BlockSpec((BE, H), lambda i: (i, 0)),
        out_shape=jax.ShapeDtypeStruct((EPAD, H), jnp.float32),
    )(ea_p, W, b.reshape(1, H), g.reshape(1, H), bb.reshape(1, H))


def _dinv_from_ptr(ptr2_blk):
    deg = (ptr2_blk[:, 1:2] - ptr2_blk[:, 0:1] + 1).astype(jnp.float32)
    return lax.rsqrt(deg)


def _tc_gcn_pre(h, ptr2, W):
    def body(h_ref, p_ref, W_ref, o_ref):
        dinv = _dinv_from_ptr(p_ref[...])
        o_ref[...] = _dot(h_ref[...], W_ref[...]) * dinv

    return pl.pallas_call(
        body,
        grid=(NPAD // BN,),
        in_specs=[pl.BlockSpec((BN, H), lambda i: (i, 0)),
                  pl.BlockSpec((BN, 2), lambda i: (i, 0)),
                  _full((H, H))],
        out_specs=pl.BlockSpec((BN, H), lambda i: (i, 0)),
        out_shape=jax.ShapeDtypeStruct((NPAD, H), jnp.float32),
    )(h, ptr2, W)


def _tc_gcn_post(S0, S1, y, ptr2, b, residual):
    def body(s0_ref, s1_ref, y_ref, p_ref, b_ref, o_ref):
        dinv = _dinv_from_ptr(p_ref[...])
        t = dinv * (s0_ref[...] + s1_ref[...] + y_ref[...]) + b_ref[...]
        r = jnp.maximum(t, 0.0)
        o_ref[...] = r + t if residual else r

    return pl.pallas_call(
        body,
        grid=(NPAD // BN,),
        in_specs=[pl.BlockSpec((BN, H), lambda i: (i, 0))] * 3 +
                 [pl.BlockSpec((BN, 2), lambda i: (i, 0)), _full((1, H))],
        out_specs=pl.BlockSpec((BN, H), lambda i: (i, 0)),
        out_shape=jax.ShapeDtypeStruct((NPAD, H), jnp.float32),
    )(S0, S1, y, ptr2, b.reshape(1, H))


def _tc_gat_pre(h, W, Ms, Md):
    def body(h_ref, W_ref, Ms_ref, Md_ref, t_ref, td_ref):
        xw = _dot(h_ref[...], W_ref[...])
        as_ = _dot(xw, Ms_ref[...])
        ad_ = _dot(xw, Md_ref[...])
        t_ref[...] = jnp.concatenate([xw, as_, ad_], axis=1)
        td_ref[...] = jnp.concatenate([ad_, jnp.zeros_like(ad_)], axis=1)

    return pl.pallas_call(
        body,
        grid=(NPAD // BN,),
        in_specs=[pl.BlockSpec((BN, H), lambda i: (i, 0)),
                  _full((H, H)), _full((H, HEADS)), _full((H, HEADS))],
        out_specs=[pl.BlockSpec((BN, H + 2 * HEADS), lambda i: (i, 0)),
                   pl.BlockSpec((BN, 2 * HEADS), lambda i: (i, 0))],
        out_shape=[jax.ShapeDtypeStruct((NPAD, H + 2 * HEADS), jnp.float32),
                   jax.ShapeDtypeStruct((NPAD, 2 * HEADS), jnp.float32)],
    )(h, W, Ms, Md)


def _tc_gat_edge(G, Gd):
    def body(g_ref, gd_ref, o_ref):
        R = _expand_mat()
        g = g_ref[...]
        e = g[:, H:H + HEADS] + gd_ref[...][:, 0:HEADS]
        e = jnp.where(e >= 0, e, 0.2 * e)
        ex = jnp.exp(e)
        num = g[:, 0:H] * _dot(ex, R)
        o_ref[...] = jnp.concatenate(
            [num, ex, jnp.zeros_like(ex)], axis=1)

    return pl.pallas_call(
        body,
        grid=(EPAD // BE,),
        in_specs=[pl.BlockSpec((BE, H + 2 * HEADS), lambda i: (i, 0)),
                  pl.BlockSpec((BE, 2 * HEADS), lambda i: (i, 0))],
        out_specs=pl.BlockSpec((BE, H + 2 * HEADS), lambda i: (i, 0)),
        out_shape=jax.ShapeDtypeStruct((EPAD, H + 2 * HEADS), jnp.float32),
    )(G, Gd)


def _tc_gat_post(S0, S1, T, b, residual):
    def body(s0_ref, s1_ref, t_ref, b_ref, o_ref):
        R = _expand_mat()
        s = s0_ref[...] + s1_ref[...]
        tt = t_ref[...]
        xw = tt[:, 0:H]
        es = tt[:, H:H + HEADS] + tt[:, H + HEADS:H + 2 * HEADS]
        es = jnp.where(es >= 0, es, 0.2 * es)
        exs = jnp.exp(es)
        num = s[:, 0:H] + xw * _dot(exs, R)
        den = _dot(s[:, H:H + HEADS] + exs, R)
        t = num / jnp.maximum(den, 1e-16) + b_ref[...]
        r = jnp.maximum(t, 0.0)
        o_ref[...] = r + t if residual else r

    return pl.pallas_call(
        body,
        grid=(NPAD // BN,),
        in_specs=[pl.BlockSpec((BN, H + 2 * HEADS), lambda i: (i, 0))] * 2 +
                 [pl.BlockSpec((BN, H + 2 * HEADS), lambda i: (i, 0)),
                  _full((1, H))],
        out_specs=pl.BlockSpec((BN, H), lambda i: (i, 0)),
        out_shape=jax.ShapeDtypeStruct((NPAD, H), jnp.float32),
    )(S0, S1, T, b.reshape(1, H))


def _tc_ec_edge(Hs, Hd, W1a, W1b, b1, W2, b2):
    def body(hs_ref, hd_ref, w1a, w1b, b1_ref, w2, b2_ref, o_ref):
        xi = hd_ref[...]
        xj = hs_ref[...]
        t = _dot(xi, w1a[...]) + _dot(xj - xi, w1b[...]) + b1_ref[...]
        t = jnp.maximum(t, 0.0)
        o_ref[...] = _dot(t, w2[...]) + b2_ref[...]

    return pl.pallas_call(
        body,
        grid=(EPAD // BE,),
        in_specs=[pl.BlockSpec((BE, H), lambda i: (i, 0))] * 2 +
                 [_full((H, H)), _full((H, H)), _full((1, H)),
                  _full((H, H)), _full((1, H))],
        out_specs=pl.BlockSpec((BE, H), lambda i: (i, 0)),
        out_shape=jax.ShapeDtypeStruct((EPAD, H), jnp.float32),
    )(Hs, Hd, W1a, W1b, b1.reshape(1, H), W2, b2.reshape(1, H))


def _tc_ec_post(M, residual):
    def body(m_ref, o_ref):
        m = m_ref[...]
        t = jnp.where(jnp.isfinite(m), m, 0.0)
        r = jnp.maximum(t, 0.0)
        o_ref[...] = r + t if residual else r

    return pl.pallas_call(
        body,
        grid=(NPAD // BN,),
        in_specs=[pl.BlockSpec((BN, H), lambda i: (i, 0))],
        out_specs=pl.BlockSpec((BN, H), lambda i: (i, 0)),
        out_shape=jax.ShapeDtypeStruct((NPAD, H), jnp.float32),
    )(M)


def _tc_global(h, gc_W1, gc_b1, gc_W2, gc_b2, ep_W1d, ep_b1):
    def body(h_ref, w1, b1, w2, b2, wd, bd, o_ref):
        rid = lax.broadcasted_iota(jnp.int32, (NPAD, 1), 0)
        hm = jnp.where(rid < N, h_ref[...], 0.0)
        g = jnp.sum(hm, axis=0, keepdims=True) / jnp.float32(N)
        g = jnp.maximum(_dot(g, w1[...]) + b1[...], 0.0)
        g = _dot(g, w2[...]) + b2[...]
        o_ref[...] = _dot(g, wd[...]) + bd[...]

    return pl.pallas_call(
        body,
        in_specs=[_full((NPAD, H)), _full((H, H)), _full((1, H)),
                  _full((H, H)), _full((1, H)), _full((H, 2 * H)),
                  _full((1, 2 * H))],
        out_specs=_full((1, 2 * H)),
        out_shape=jax.ShapeDtypeStruct((1, 2 * H), jnp.float32),
    )(h, gc_W1, gc_b1.reshape(1, H), gc_W2, gc_b2.reshape(1, H),
      ep_W1d, ep_b1.reshape(1, 2 * H))


def _tc_pred(Ps, Pd, eh, cbias, W1a, W1b, W1c, ln1g, ln1b,
             W2, b2, ln2g, ln2b, W3, b3):
    def body(ps_ref, pd_ref, eh_ref, cb, w1a, w1b, w1c, g1, bb1,
             w2, bb2, g2, bb2n, w3, bb3, o_ref):
        z = (_dot(ps_ref[...], w1a[...]) + _dot(pd_ref[...], w1b[...]) +
             _dot(eh_ref[...], w1c[...]) + cb[...])
        z = jnp.maximum(z, 0.0)
        z = _ln(z, g1[...], bb1[...])
        z = jnp.maximum(_dot(z, w2[...]) + bb2[...], 0.0)
        z = _ln(z, g2[...], bb2n[...])
        o_ref[...] = _dot(z, w3[...]) + bb3[...]

    return pl.pallas_call(
        body,
        grid=(EPAD // BE,),
        in_specs=[pl.BlockSpec((BE, H), lambda i: (i, 0))] * 3 +
                 [_full((1, 2 * H)), _full((H, 2 * H)), _full((H, 2 * H)),
                  _full((H, 2 * H)), _full((1, 2 * H)), _full((1, 2 * H)),
                  _full((2 * H, H)), _full((1, H)), _full((1, H)),
                  _full((1, H)), _full((H, 8)), _full((1, 8))],
        out_specs=pl.BlockSpec((BE, 8), lambda i: (i, 0)),
        out_shape=jax.ShapeDtypeStruct((EPAD, 8), jnp.float32),
    )(Ps, Pd, eh, cbias, W1a, W1b, W1c, ln1g.reshape(1, 2 * H),
      ln1b.reshape(1, 2 * H), W2, b2.reshape(1, H), ln2g.reshape(1, H),
      ln2b.reshape(1, H), W3, b3.reshape(1, 8))


def _gather64(t, i): return _sc_gather(H)(t, i)
def _gather80(t, i): return _sc_gather(H + 2 * HEADS)(t, i)
def _gather16(t, i): return _sc_gather(2 * HEADS)(t, i)
def _scatter64(v, i, z): return _sc_scatter_add(H)(v, i, z)
def _scatter80(v, i, z): return _sc_scatter_add(H + 2 * HEADS)(v, i, z)
def _segmax(m, d, b): return _sc_segmax()(m, d, b)


def kernel(x, edge_index, edge_attr, params):
    p = params
    src = edge_index[0].astype(jnp.int32)
    dst = edge_index[1].astype(jnp.int32)

    perm = jnp.argsort(dst)
    dst_s = dst[perm]
    src_s = src[perm]
    ptr = jnp.searchsorted(dst_s, jnp.arange(N + 1, dtype=jnp.int32)
                           ).astype(jnp.int32)
    ptr2 = jnp.zeros((NPAD, 2), jnp.int32)
    ptr2 = ptr2.at[:N, 0].set(ptr[:N]).at[:N, 1].set(ptr[1:])
    bnd = jnp.searchsorted(
        dst_s, (jnp.arange(33, dtype=jnp.int32) * NPT).astype(jnp.int32)
    ).astype(jnp.int32)
    bnd = jnp.pad(bnd, (0, 15), constant_values=E)

    pad_e = EPAD - E
    dst_sp = jnp.pad(dst_s, (0, pad_e), constant_values=NPAD - 1)
    src_sp = jnp.pad(src_s, (0, pad_e), constant_values=0)
    src_p = jnp.pad(src, (0, pad_e), constant_values=0)
    dst_p = jnp.pad(dst, (0, pad_e), constant_values=0)

    x_p = jnp.pad(x, ((0, NPAD - N), (0, 0)))
    ea_p = jnp.pad(edge_attr, ((0, pad_e), (0, 0)))

    z64 = jnp.zeros((NPS, H), jnp.float32)
    z80 = jnp.zeros((NPS, H + 2 * HEADS), jnp.float32)

    eyeh = jnp.eye(HEADS, dtype=jnp.float32)
    Ms0 = {}
    for j in range(2):
        for nm in ('as', 'ad'):
            a = p[f'gat_{nm}_{j}']
            Ms0[(nm, j)] = (a[:, :, None] * eyeh[:, None, :]).reshape(H, HEADS)

    h = _tc_enc(x_p, p['enc_W'], p['enc_b'], p['enc_bn_g'], p['enc_bn_b'])
    eh = _tc_eh(ea_p, p['ee_W'], p['ee_b'], p['ee_ln_g'], p['ee_ln_b'])

    for i in range(6):
        j = i // 3
        residual = (i % 2 == 1)
        if i % 3 == 0:
            y = _tc_gcn_pre(h, ptr2, p[f'gcn_W_{j}'])
            g = _gather64(y, src_sp)
            S = _scatter64(g, dst_sp, z64)
            h = _tc_gcn_post(S[0], S[1], y, ptr2, p[f'gcn_b_{j}'], residual)
        elif i % 3 == 1:
            T, Td = _tc_gat_pre(h, p[f'gat_W_{j}'],
                                Ms0[('as', j)], Ms0[('ad', j)])
            G = _gather80(T, src_sp)
            Gd = _gather16(Td, dst_sp)
            numex = _tc_gat_edge(G, Gd)
            S = _scatter80(numex, dst_sp, z80)
            h = _tc_gat_post(S[0], S[1], T, p[f'gat_b_{j}'], residual)
        else:
            Hs = _gather64(h, src_sp)
            Hd = _gather64(h, dst_sp)
            W1 = p[f'ec_W1_{j}']
            m = _tc_ec_edge(Hs, Hd, W1[:H], W1[H:], p[f'ec_b1_{j}'],
                            p[f'ec_W2_{j}'], p[f'ec_b2_{j}'])
            M = _segmax(m.reshape(EPAD * H), dst_sp, bnd)
            h = _tc_ec_post(M.reshape(NPAD, H), residual)

    ep_W1 = p['ep_W1']
    cbias = _tc_global(h, p['gc_W1'], p['gc_b1'], p['gc_W2'], p['gc_b2'],
                       ep_W1[3 * H:], p['ep_b1'])

    Ps = _gather64(h, src_p)
    Pd = _gather64(h, dst_p)
    W3 = jnp.pad(p['ep_W3'], ((0, 0), (0, 7)))
    b3 = jnp.pad(p['ep_b3'], (0, 7))
    logits = _tc_pred(Ps, Pd, eh, cbias,
                      ep_W1[:H], ep_W1[H:2 * H], ep_W1[2 * H:3 * H],
                      p['ep_ln1_g'], p['ep_ln1_b'],
                      p['ep_W2'], p['ep_b2'], p['ep_ln2_g'], p['ep_ln2_b'],
                      W3, b3)
    return logits[:E, 0]


# speedup vs baseline: 11.4479x; 1.4938x over previous
import functools

import jax
import jax.numpy as jnp
from jax import lax
from jax.experimental import pallas as pl
from jax.experimental.pallas import tpu as pltpu
from jax.experimental.pallas import tpu_sc as plsc

N = 10000
E = 320000
D_IN = 128
H = 64
HEADS = 8
CH = H // HEADS

NW = 32
NPT = 313
NPAD = NW * NPT
C = 128
EPW = 79 * C
EPAD = EPW * NW
BE = 2048
BN = 2504
NPS = NPAD // 16

@functools.lru_cache(maxsize=None)
def _mesh():
    return plsc.VectorSubcoreMesh(core_axis_name="c", subcore_axis_name="s")


@functools.lru_cache(maxsize=None)
def _sc_gather(D):
    @functools.partial(
        pl.kernel,
        out_type=jax.ShapeDtypeStruct((EPAD, D), jnp.float32),
        mesh=_mesh(),
        scratch_types=[
            pltpu.VMEM((C,), jnp.int32),
            pltpu.VMEM((C, D), jnp.float32),
            pltpu.SemaphoreType.DMA,
        ],
        name=f"sc_gather_{D}",
        compiler_params=pltpu.CompilerParams(use_tc_tiling_on_sc=False),
    )
    def k(table, idx, out, idx_v, rows_v, sem):
        w = lax.axis_index("s") * 2 + lax.axis_index("c")
        base = w * EPW

        def body(i, carry):
            off = base + i * C
            pltpu.sync_copy(idx.at[pl.ds(off, C)], idx_v)
            pltpu.async_copy(table.at[idx_v], rows_v, sem).wait()
            pltpu.sync_copy(rows_v, out.at[pl.ds(off, C)])
            return carry

        lax.fori_loop(0, EPW // C, body, 0)

    return k


@functools.lru_cache(maxsize=None)
def _sc_scatter_add(D):
    @functools.partial(
        pl.kernel,
        out_type=jax.ShapeDtypeStruct((2, NPAD, D), jnp.float32),
        mesh=_mesh(),
        scratch_types=[
            pltpu.VMEM((C,), jnp.int32),
            pltpu.VMEM((C, D), jnp.float32),
            pltpu.VMEM((NPS, D), jnp.float32),
            pltpu.VMEM_SHARED((NPAD, D), jnp.float32),
        ],
        name=f"sc_scatter_add_{D}",
        compiler_params=pltpu.CompilerParams(use_tc_tiling_on_sc=False),
    )
    def k(vals, idx, zeros, out, idx_v, vals_v, zrow_v, acc):
        c = lax.axis_index("c")
        s = lax.axis_index("s")
        w = s * 2 + c
        pltpu.sync_copy(zeros, zrow_v)
        pltpu.sync_copy(zrow_v, acc.at[pl.ds(s * NPS, NPS)])
        plsc.subcore_barrier()

        base = w * EPW

        def body(i, carry):
            off = base + i * C
            pltpu.sync_copy(idx.at[pl.ds(off, C)], idx_v)
            pltpu.sync_copy(vals.at[pl.ds(off, C)], vals_v)
            pltpu.sync_copy(vals_v, acc.at[idx_v], add=True)
            return carry

        lax.fori_loop(0, EPW // C, body, 0)
        plsc.subcore_barrier()
        pltpu.sync_copy(acc.at[pl.ds(s * NPS, NPS)],
                        out.at[c].at[pl.ds(s * NPS, NPS)])

    return k


@functools.lru_cache(maxsize=None)
def _sc_segmax():
    NEG = jnp.float32(-jnp.inf)

    @functools.partial(
        pl.kernel,
        out_type=jax.ShapeDtypeStruct((NPAD * H,), jnp.float32),
        mesh=_mesh(),
        scratch_types=[
            pltpu.VMEM((48,), jnp.int32),
            pltpu.VMEM((C,), jnp.int32),
            pltpu.VMEM((C * H,), jnp.float32),
            pltpu.VMEM((NPT * H,), jnp.float32),
        ],
        name="sc_segmax",
        compiler_params=pltpu.CompilerParams(use_tc_tiling_on_sc=False),
    )
    def k(m_flat, dsts, bnd, out, bnd_v, dst_v, vals_v, acc):
        w = lax.axis_index("s") * 2 + lax.axis_index("c")
        base = w * NPT

        def init(i, carry):
            acc[pl.ds(i * 16, 16)] = jnp.full((16,), NEG, jnp.float32)
            return carry

        lax.fori_loop(0, NPT * H // 16, init, 0)

        pltpu.sync_copy(bnd, bnd_v)
        bv = bnd_v[pl.ds(w, 16)]
        start = bv[0]
        end = bv[1]
        c0 = (start // C) * C
        nch = (end - c0 + C - 1) // C
        nch = jnp.maximum(nch, 0)

        def chunk(kk, carry):
            off = c0 + kk * C
            pltpu.sync_copy(m_flat.at[pl.ds(off * H, C * H)], vals_v)
            pltpu.sync_copy(dsts.at[pl.ds(off, C)], dst_v)

            def grp(gg, cc2):
                dvec = dst_v[pl.ds(gg * 16, 16)] - base
                for lane in range(16):
                    e = off + gg * 16 + lane
                    valid = jnp.logical_and(e >= start, e < end)
                    dloc = jnp.where(valid, dvec[lane], 0)
                    abase = dloc * H
                    vbase = (gg * 16 + lane) * H
                    for cc in range(H // 16):
                        v = vals_v[pl.ds(vbase + cc * 16, 16)]
                        v = jnp.where(valid, v,
                                      jnp.full((16,), NEG, jnp.float32))
                        a = acc[pl.ds(abase + cc * 16, 16)]
                        acc[pl.ds(abase + cc * 16, 16)] = jnp.maximum(a, v)
                return cc2

            lax.fori_loop(0, C // 16, grp, 0)
            return carry

        lax.fori_loop(0, nch, chunk, 0)
        pltpu.sync_copy(acc, out.at[pl.ds(base * H, NPT * H)])

    return k


def _dot(a, b):
    return jnp.dot(a, b, preferred_element_type=jnp.float32)


def _full(shape):
    return pl.BlockSpec(shape, lambda *_: (0,) * len(shape))


def _ln(z, g, b, eps=1e-5):
    mu = jnp.mean(z, axis=-1, keepdims=True)
    va = jnp.mean((z - mu) * (z - mu), axis=-1, keepdims=True)
    return (z - mu) / jnp.sqrt(va + eps) * g + b


def _expand_mat():
    hh = lax.broadcasted_iota(jnp.int32, (HEADS, H), 0)
    jj = lax.broadcasted_iota(jnp.int32, (HEADS, H), 1)
    return (jj // CH == hh).astype(jnp.float32)


def _tc_enc(x_p, W, b, g, bb):
    def body(x_ref, W_ref, b_ref, g_ref, bb_ref, o_ref):
        t = _dot(x_ref[...], W_ref[...]) + b_ref[...]
        t = t / jnp.sqrt(jnp.float32(1.0 + 1e-5)) * g_ref[...] + bb_ref[...]
        o_ref[...] = jnp.maximum(t, 0.0)

    return pl.pallas_call(
        body,
        grid=(NPAD // BN,),
        in_specs=[pl.BlockSpec((BN, D_IN), lambda i: (i, 0)),
                  _full((D_IN, H)), _full((1, H)), _full((1, H)), _full((1, H))],
        out_specs=pl.BlockSpec((BN, H), lambda i: (i, 0)),
        out_shape=jax.ShapeDtypeStruct((NPAD, H), jnp.float32),
    )(x_p, W, b.reshape(1, H), g.reshape(1, H), bb.reshape(1, H))


def _tc_eh(ea_p, W, b, g, bb):
    def body(a_ref, W_ref, b_ref, g_ref, bb_ref, o_ref):
        t = jnp.maximum(_dot(a_ref[...], W_ref[...]) + b_ref[...], 0.0)
        o_ref[...] = _ln(t, g_ref[...], bb_ref[...])

    return pl.pallas_call(
        body,
        grid=(EPAD // BE,),
        in_specs=[pl.BlockSpec((BE, 16), lambda i: (i, 0)),
                  _full((16, H)), _full((1, H)), _full((1, H)), _full((1, H))],
        out_specs=pl.BlockSpec((BE, H), lambda i: (i, 0)),
        out_shape=jax.ShapeDtypeStruct((EPAD, H), jnp.float32),
    )(ea_p, W, b.reshape(1, H), g.reshape(1, H), bb.reshape(1, H))


def _dinv_from_ptr(ptr2_blk):
    deg = (ptr2_blk[:, 1:2] - ptr2_blk[:, 0:1] + 1).astype(jnp.float32)
    return lax.rsqrt(deg)


def _tc_gcn_pre(h, ptr2, W):
    def body(h_ref, p_ref, W_ref, o_ref):
        dinv = _dinv_from_ptr(p_ref[...])
        o_ref[...] = _dot(h_ref[...], W_ref[...]) * dinv

    return pl.pallas_call(
        body,
        grid=(NPAD // BN,),
        in_specs=[pl.BlockSpec((BN, H), lambda i: (i, 0)),
                  pl.BlockSpec((BN, 2), lambda i: (i, 0)),
                  _full((H, H))],
        out_specs=pl.BlockSpec((BN, H), lambda i: (i, 0)),
        out_shape=jax.ShapeDtypeStruct((NPAD, H), jnp.float32),
    )(h, ptr2, W)


def _tc_gcn_post(S0, S1, y, ptr2, b, residual):
    def body(s0_ref, s1_ref, y_ref, p_ref, b_ref, o_ref):
        dinv = _dinv_from_ptr(p_ref[...])
        t = dinv * (s0_ref[...] + s1_ref[...] + y_ref[...]) + b_ref[...]
        r = jnp.maximum(t, 0.0)
        o_ref[...] = r + t if residual else r

    return pl.pallas_call(
        body,
        grid=(NPAD // BN,),
        in_specs=[pl.BlockSpec((BN, H), lambda i: (i, 0))] * 3 +
                 [pl.BlockSpec((BN, 2), lambda i: (i, 0)), _full((1, H))],
        out_specs=pl.BlockSpec((BN, H), lambda i: (i, 0)),
        out_shape=jax.ShapeDtypeStruct((NPAD, H), jnp.float32),
    )(S0, S1, y, ptr2, b.reshape(1, H))


def _tc_gat_pre(h, W, Ms, Md):
    def body(h_ref, W_ref, Ms_ref, Md_ref, t_ref, td_ref):
        xw = _dot(h_ref[...], W_ref[...])
        as_ = _dot(xw, Ms_ref[...])
        ad_ = _dot(xw, Md_ref[...])
        t_ref[...] = jnp.concatenate([xw, as_, ad_], axis=1)
        td_ref[...] = jnp.concatenate([ad_, jnp.zeros_like(ad_)], axis=1)

    return pl.pallas_call(
        body,
        grid=(NPAD // BN,),
        in_specs=[pl.BlockSpec((BN, H), lambda i: (i, 0)),
                  _full((H, H)), _full((H, HEADS)), _full((H, HEADS))],
        out_specs=[pl.BlockSpec((BN, H + 2 * HEADS), lambda i: (i, 0)),
                   pl.BlockSpec((BN, 2 * HEADS), lambda i: (i, 0))],
        out_shape=[jax.ShapeDtypeStruct((NPAD, H + 2 * HEADS), jnp.float32),
                   jax.ShapeDtypeStruct((NPAD, 2 * HEADS), jnp.float32)],
    )(h, W, Ms, Md)


def _tc_gat_edge(G, Gd):
    def body(g_ref, gd_ref, o_ref):
        R = _expand_mat()
        g = g_ref[...]
        e = g[:, H:H + HEADS] + gd_ref[...][:, 0:HEADS]
        e = jnp.where(e >= 0, e, 0.2 * e)
        ex = jnp.exp(e)
        num = g[:, 0:H] * _dot(ex, R)
        o_ref[...] = jnp.concatenate(
            [num, ex, jnp.zeros_like(ex)], axis=1)

    return pl.pallas_call(
        body,
        grid=(EPAD // BE,),
        in_specs=[pl.BlockSpec((BE, H + 2 * HEADS), lambda i: (i, 0)),
                  pl.BlockSpec((BE, 2 * HEADS), lambda i: (i, 0))],
        out_specs=pl.BlockSpec((BE, H + 2 * HEADS), lambda i: (i, 0)),
        out_shape=jax.ShapeDtypeStruct((EPAD, H + 2 * HEADS), jnp.float32),
    )(G, Gd)


def _tc_gat_post(S0, S1, T, b, residual):
    def body(s0_ref, s1_ref, t_ref, b_ref, o_ref):
        R = _expand_mat()
        s = s0_ref[...] + s1_ref[...]
        tt = t_ref[...]
        xw = tt[:, 0:H]
        es = tt[:, H:H + HEADS] + tt[:, H + HEADS:H + 2 * HEADS]
        es = jnp.where(es >= 0, es, 0.2 * es)
        exs = jnp.exp(es)
        num = s[:, 0:H] + xw * _dot(exs, R)
        den = _dot(s[:, H:H + HEADS] + exs, R)
        t = num / jnp.maximum(den, 1e-16) + b_ref[...]
        r = jnp.maximum(t, 0.0)
        o_ref[...] = r + t if residual else r

    return pl.pallas_call(
        body,
        grid=(NPAD // BN,),
        in_specs=[pl.BlockSpec((BN, H + 2 * HEADS), lambda i: (i, 0))] * 2 +
                 [pl.BlockSpec((BN, H + 2 * HEADS), lambda i: (i, 0)),
                  _full((1, H))],
        out_specs=pl.BlockSpec((BN, H), lambda i: (i, 0)),
        out_shape=jax.ShapeDtypeStruct((NPAD, H), jnp.float32),
    )(S0, S1, T, b.reshape(1, H))


def _tc_ec_edge(Hs, Hd, W1a, W1b, b1, W2, b2):
    def body(hs_ref, hd_ref, w1a, w1b, b1_ref, w2, b2_ref, o_ref):
        xi = hd_ref[...]
        xj = hs_ref[...]
        t = _dot(xi, w1a[...]) + _dot(xj - xi, w1b[...]) + b1_ref[...]
        t = jnp.maximum(t, 0.0)
        o_ref[...] = _dot(t, w2[...]) + b2_ref[...]

    return pl.pallas_call(
        body,
        grid=(EPAD // BE,),
        in_specs=[pl.BlockSpec((BE, H), lambda i: (i, 0))] * 2 +
                 [_full((H, H)), _full((H, H)), _full((1, H)),
                  _full((H, H)), _full((1, H))],
        out_specs=pl.BlockSpec((BE, H), lambda i: (i, 0)),
        out_shape=jax.ShapeDtypeStruct((EPAD, H), jnp.float32),
    )(Hs, Hd, W1a, W1b, b1.reshape(1, H), W2, b2.reshape(1, H))


def _tc_ec_post(M, residual):
    def body(m_ref, o_ref):
        m = m_ref[...]
        t = jnp.where(jnp.isfinite(m), m, 0.0)
        r = jnp.maximum(t, 0.0)
        o_ref[...] = r + t if residual else r

    return pl.pallas_call(
        body,
        grid=(NPAD // BN,),
        in_specs=[pl.BlockSpec((BN, H), lambda i: (i, 0))],
        out_specs=pl.BlockSpec((BN, H), lambda i: (i, 0)),
        out_shape=jax.ShapeDtypeStruct((NPAD, H), jnp.float32),
    )(M)


def _tc_global(h, gc_W1, gc_b1, gc_W2, gc_b2, ep_W1d, ep_b1):
    def body(h_ref, w1, b1, w2, b2, wd, bd, o_ref):
        rid = lax.broadcasted_iota(jnp.int32, (NPAD, 1), 0)
        hm = jnp.where(rid < N, h_ref[...], 0.0)
        g = jnp.sum(hm, axis=0, keepdims=True) / jnp.float32(N)
        g = jnp.maximum(_dot(g, w1[...]) + b1[...], 0.0)
        g = _dot(g, w2[...]) + b2[...]
        o_ref[...] = _dot(g, wd[...]) + bd[...]

    return pl.pallas_call(
        body,
        in_specs=[_full((NPAD, H)), _full((H, H)), _full((1, H)),
                  _full((H, H)), _full((1, H)), _full((H, 2 * H)),
                  _full((1, 2 * H))],
        out_specs=_full((1, 2 * H)),
        out_shape=jax.ShapeDtypeStruct((1, 2 * H), jnp.float32),
    )(h, gc_W1, gc_b1.reshape(1, H), gc_W2, gc_b2.reshape(1, H),
      ep_W1d, ep_b1.reshape(1, 2 * H))


def _tc_pred(Ps, Pd, eh, cbias, W1a, W1b, W1c, ln1g, ln1b,
             W2, b2, ln2g, ln2b, W3, b3):
    def body(ps_ref, pd_ref, eh_ref, cb, w1a, w1b, w1c, g1, bb1,
             w2, bb2, g2, bb2n, w3, bb3, o_ref):
        z = (_dot(ps_ref[...], w1a[...]) + _dot(pd_ref[...], w1b[...]) +
             _dot(eh_ref[...], w1c[...]) + cb[...])
        z = jnp.maximum(z, 0.0)
        z = _ln(z, g1[...], bb1[...])
        z = jnp.maximum(_dot(z, w2[...]) + bb2[...], 0.0)
        z = _ln(z, g2[...], bb2n[...])
        o_ref[...] = _dot(z, w3[...]) + bb3[...]

    return pl.pallas_call(
        body,
        grid=(EPAD // BE,),
        in_specs=[pl.BlockSpec((BE, H), lambda i: (i, 0))] * 3 +
                 [_full((1, 2 * H)), _full((H, 2 * H)), _full((H, 2 * H)),
                  _full((H, 2 * H)), _full((1, 2 * H)), _full((1, 2 * H)),
                  _full((2 * H, H)), _full((1, H)), _full((1, H)),
                  _full((1, H)), _full((H, 8)), _full((1, 8))],
        out_specs=pl.BlockSpec((BE, 8), lambda i: (i, 0)),
        out_shape=jax.ShapeDtypeStruct((EPAD, 8), jnp.float32),
    )(Ps, Pd, eh, cbias, W1a, W1b, W1c, ln1g.reshape(1, 2 * H),
      ln1b.reshape(1, 2 * H), W2, b2.reshape(1, H), ln2g.reshape(1, H),
      ln2b.reshape(1, H), W3, b3.reshape(1, 8))


def _gather64(t, i): return _sc_gather(H)(t, i)
def _gather80(t, i): return _sc_gather(H + 2 * HEADS)(t, i)
def _gather16(t, i): return _sc_gather(2 * HEADS)(t, i)
def _scatter64(v, i, z): return _sc_scatter_add(H)(v, i, z)
def _scatter80(v, i, z): return _sc_scatter_add(H + 2 * HEADS)(v, i, z)
def _segmax(m, d, b): return _sc_segmax()(m, d, b)


def kernel(x, edge_index, edge_attr, params):
    p = params
    src = edge_index[0].astype(jnp.int32)
    dst = edge_index[1].astype(jnp.int32)

    key = jnp.sort(dst * 32768 + src)
    dst_s = key >> 15
    src_s = key & 32767
    ptr = jnp.searchsorted(key, jnp.arange(N + 1, dtype=jnp.int32) * 32768
                           ).astype(jnp.int32)
    ptr2 = jnp.zeros((NPAD, 2), jnp.int32)
    ptr2 = ptr2.at[:N, 0].set(ptr[:N]).at[:N, 1].set(ptr[1:])
    bnd = ptr[jnp.minimum(jnp.arange(33, dtype=jnp.int32) * NPT, N)]
    bnd = jnp.pad(bnd, (0, 15), constant_values=E)

    pad_e = EPAD - E
    dst_sp = jnp.pad(dst_s, (0, pad_e), constant_values=NPAD - 1)
    src_sp = jnp.pad(src_s, (0, pad_e), constant_values=0)
    src_p = jnp.pad(src, (0, pad_e), constant_values=0)
    dst_p = jnp.pad(dst, (0, pad_e), constant_values=0)

    x_p = jnp.pad(x, ((0, NPAD - N), (0, 0)))
    ea_p = jnp.pad(edge_attr, ((0, pad_e), (0, 0)))

    z64 = jnp.zeros((NPS, H), jnp.float32)
    z80 = jnp.zeros((NPS, H + 2 * HEADS), jnp.float32)

    eyeh = jnp.eye(HEADS, dtype=jnp.float32)
    Ms0 = {}
    for j in range(2):
        for nm in ('as', 'ad'):
            a = p[f'gat_{nm}_{j}']
            Ms0[(nm, j)] = (a[:, :, None] * eyeh[:, None, :]).reshape(H, HEADS)

    h = _tc_enc(x_p, p['enc_W'], p['enc_b'], p['enc_bn_g'], p['enc_bn_b'])
    eh = _tc_eh(ea_p, p['ee_W'], p['ee_b'], p['ee_ln_g'], p['ee_ln_b'])

    for i in range(6):
        j = i // 3
        residual = (i % 2 == 1)
        if i % 3 == 0:
            y = _tc_gcn_pre(h, ptr2, p[f'gcn_W_{j}'])
            g = _gather64(y, src_sp)
            S = _scatter64(g, dst_sp, z64)
            h = _tc_gcn_post(S[0], S[1], y, ptr2, p[f'gcn_b_{j}'], residual)
        elif i % 3 == 1:
            T, Td = _tc_gat_pre(h, p[f'gat_W_{j}'],
                                Ms0[('as', j)], Ms0[('ad', j)])
            G = _gather80(T, src_sp)
            Gd = _gather16(Td, dst_sp)
            numex = _tc_gat_edge(G, Gd)
            S = _scatter80(numex, dst_sp, z80)
            h = _tc_gat_post(S[0], S[1], T, p[f'gat_b_{j}'], residual)
        else:
            Hs = _gather64(h, src_sp)
            Hd = _gather64(h, dst_sp)
            W1 = p[f'ec_W1_{j}']
            m = _tc_ec_edge(Hs, Hd, W1[:H], W1[H:], p[f'ec_b1_{j}'],
                            p[f'ec_W2_{j}'], p[f'ec_b2_{j}'])
            M = _segmax(m.reshape(EPAD * H), dst_sp, bnd)
            h = _tc_ec_post(M.reshape(NPAD, H), residual)

    ep_W1 = p['ep_W1']
    cbias = _tc_global(h, p['gc_W1'], p['gc_b1'], p['gc_W2'], p['gc_b2'],
                       ep_W1[3 * H:], p['ep_b1'])

    Ps = _gather64(h, src_p)
    Pd = _gather64(h, dst_p)
    W3 = jnp.pad(p['ep_W3'], ((0, 0), (0, 7)))
    b3 = jnp.pad(p['ep_b3'], (0, 7))
    logits = _tc_pred(Ps, Pd, eh, cbias,
                      ep_W1[:H], ep_W1[H:2 * H], ep_W1[2 * H:3 * H],
                      p['ep_ln1_g'], p['ep_ln1_b'],
                      p['ep_W2'], p['ep_b2'], p['ep_ln2_g'], p['ep_ln2_b'],
                      W3, b3)
    return logits[:E, 0]


# speedup vs baseline: 11.5513x; 1.0090x over previous
import functools

import jax
import jax.numpy as jnp
from jax import lax
from jax.experimental import pallas as pl
from jax.experimental.pallas import tpu as pltpu
from jax.experimental.pallas import tpu_sc as plsc

N = 10000
E = 320000
D_IN = 128
H = 64
HEADS = 8
CH = H // HEADS

NW = 32
NPT = 313
NPAD = NW * NPT
C = 128
NCH = 80
EPW = NCH * C
EPAD = EPW * NW
BE = 2048
BN = 2504
NPS = NPAD // 16

@functools.lru_cache(maxsize=None)
def _mesh():
    return plsc.VectorSubcoreMesh(core_axis_name="c", subcore_axis_name="s")


def _pipe_gather(table, idx, out, base, idxv, rowsv, sems):
    def start(i, b):
        pltpu.sync_copy(idx.at[pl.ds(base + i * C, C)], idxv[b])
        pltpu.async_copy(table.at[idxv[b]], rowsv[b], sems[b])

    def fin(i, b):
        pltpu.make_async_copy(table.at[idxv[b]], rowsv[b], sems[b]).wait()
        pltpu.sync_copy(rowsv[b], out.at[pl.ds(base + i * C, C)])

    start(0, 0)

    def body(g, carry):
        i0 = 2 * g
        i1 = i0 + 1
        start(i1, 1)
        fin(i0, 0)

        @pl.when(i1 + 1 < NCH)
        def _():
            start(i1 + 1, 0)

        fin(i1, 1)
        return carry

    lax.fori_loop(0, NCH // 2, body, 0)


@functools.lru_cache(maxsize=None)
def _sc_gather_pair(D1, D2):
    @functools.partial(
        pl.kernel,
        out_type=(jax.ShapeDtypeStruct((EPAD, D1), jnp.float32),
                  jax.ShapeDtypeStruct((EPAD, D2), jnp.float32)),
        mesh=_mesh(),
        scratch_types=[
            pltpu.VMEM((C,), jnp.int32),
            pltpu.VMEM((C,), jnp.int32),
            pltpu.VMEM((C, D1), jnp.float32),
            pltpu.VMEM((C, D1), jnp.float32),
            pltpu.VMEM((C, D2), jnp.float32),
            pltpu.VMEM((C, D2), jnp.float32),
            pltpu.SemaphoreType.DMA,
            pltpu.SemaphoreType.DMA,
        ],
        name=f"sc_gather_pair_{D1}_{D2}",
        compiler_params=pltpu.CompilerParams(use_tc_tiling_on_sc=False),
    )
    def k(tab1, idx1, tab2, idx2, out1, out2,
          iv0, iv1, r10, r11, r20, r21, semA, semB):
        w = lax.axis_index("s") * 2 + lax.axis_index("c")
        base = w * EPW
        _pipe_gather(tab1, idx1, out1, base, (iv0, iv1), (r10, r11),
                     (semA, semB))
        _pipe_gather(tab2, idx2, out2, base, (iv0, iv1), (r20, r21),
                     (semA, semB))

    return k


@functools.lru_cache(maxsize=None)
def _sc_scatter_add(D):
    @functools.partial(
        pl.kernel,
        out_type=jax.ShapeDtypeStruct((2, NPAD, D), jnp.float32),
        mesh=_mesh(),
        scratch_types=[
            pltpu.VMEM((C,), jnp.int32),
            pltpu.VMEM((C,), jnp.int32),
            pltpu.VMEM((C, D), jnp.float32),
            pltpu.VMEM((C, D), jnp.float32),
            pltpu.VMEM((NPS, D), jnp.float32),
            pltpu.VMEM_SHARED((NPAD, D), jnp.float32),
            pltpu.SemaphoreType.DMA,
            pltpu.SemaphoreType.DMA,
        ],
        name=f"sc_scatter_add_{D}",
        compiler_params=pltpu.CompilerParams(use_tc_tiling_on_sc=False),
    )
    def k(vals, idx, zeros, out, iv0, iv1, v0, v1, zrow_v, acc, semA, semB):
        c = lax.axis_index("c")
        s = lax.axis_index("s")
        w = s * 2 + c
        pltpu.sync_copy(zeros, zrow_v)
        pltpu.sync_copy(zrow_v, acc.at[pl.ds(s * NPS, NPS)])
        plsc.subcore_barrier()

        base = w * EPW
        vv = (v0, v1)
        iv = (iv0, iv1)
        sems = (semA, semB)

        def startv(i, b):
            pltpu.async_copy(vals.at[pl.ds(base + i * C, C)], vv[b], sems[b])

        def fin(i, b):
            pltpu.sync_copy(idx.at[pl.ds(base + i * C, C)], iv[b])
            pltpu.make_async_copy(vals.at[pl.ds(base + i * C, C)], vv[b],
                                  sems[b]).wait()
            pltpu.sync_copy(vv[b], acc.at[iv[b]], add=True)

        startv(0, 0)

        def body(g, carry):
            i0 = 2 * g
            i1 = i0 + 1
            startv(i1, 1)
            fin(i0, 0)

            @pl.when(i1 + 1 < NCH)
            def _():
                startv(i1 + 1, 0)

            fin(i1, 1)
            return carry

        lax.fori_loop(0, NCH // 2, body, 0)
        plsc.subcore_barrier()
        pltpu.sync_copy(acc.at[pl.ds(s * NPS, NPS)],
                        out.at[c].at[pl.ds(s * NPS, NPS)])

    return k


@functools.lru_cache(maxsize=None)
def _sc_gcn_agg():
    @functools.partial(
        pl.kernel,
        out_type=jax.ShapeDtypeStruct((2, NPAD, H), jnp.float32),
        mesh=_mesh(),
        scratch_types=[
            pltpu.VMEM((C,), jnp.int32),
            pltpu.VMEM((C,), jnp.int32),
            pltpu.VMEM((C,), jnp.int32),
            pltpu.VMEM((C,), jnp.int32),
            pltpu.VMEM((C, H), jnp.float32),
            pltpu.VMEM((C, H), jnp.float32),
            pltpu.VMEM((NPS, H), jnp.float32),
            pltpu.VMEM_SHARED((NPAD, H), jnp.float32),
            pltpu.SemaphoreType.DMA,
            pltpu.SemaphoreType.DMA,
        ],
        name="sc_gcn_agg",
        compiler_params=pltpu.CompilerParams(use_tc_tiling_on_sc=False),
    )
    def k(table, sidx, didx, zeros, out,
          si0, si1, di0, di1, r0, r1, zrow_v, acc, semA, semB):
        c = lax.axis_index("c")
        s = lax.axis_index("s")
        w = s * 2 + c
        pltpu.sync_copy(zeros, zrow_v)
        pltpu.sync_copy(zrow_v, acc.at[pl.ds(s * NPS, NPS)])
        plsc.subcore_barrier()

        base = w * EPW
        siv = (si0, si1)
        div = (di0, di1)
        rv = (r0, r1)
        sems = (semA, semB)

        def start(i, b):
            pltpu.sync_copy(sidx.at[pl.ds(base + i * C, C)], siv[b])
            pltpu.async_copy(table.at[siv[b]], rv[b], sems[b])

        def fin(i, b):
            pltpu.sync_copy(didx.at[pl.ds(base + i * C, C)], div[b])
            pltpu.make_async_copy(table.at[siv[b]], rv[b], sems[b]).wait()
            pltpu.sync_copy(rv[b], acc.at[div[b]], add=True)

        start(0, 0)

        def body(g, carry):
            i0 = 2 * g
            i1 = i0 + 1
            start(i1, 1)
            fin(i0, 0)

            @pl.when(i1 + 1 < NCH)
            def _():
                start(i1 + 1, 0)

            fin(i1, 1)
            return carry

        lax.fori_loop(0, NCH // 2, body, 0)
        plsc.subcore_barrier()
        pltpu.sync_copy(acc.at[pl.ds(s * NPS, NPS)],
                        out.at[c].at[pl.ds(s * NPS, NPS)])

    return k


@functools.lru_cache(maxsize=None)
def _sc_segmax():
    NEG = jnp.float32(-jnp.inf)

    @functools.partial(
        pl.kernel,
        out_type=jax.ShapeDtypeStruct((NPAD * H,), jnp.float32),
        mesh=_mesh(),
        scratch_types=[
            pltpu.VMEM((48,), jnp.int32),
            pltpu.VMEM((C,), jnp.int32),
            pltpu.VMEM((C,), jnp.int32),
            pltpu.VMEM((C * H,), jnp.float32),
            pltpu.VMEM((C * H,), jnp.float32),
            pltpu.VMEM((NPT * H,), jnp.float32),
            pltpu.SemaphoreType.DMA,
            pltpu.SemaphoreType.DMA,
        ],
        name="sc_segmax",
        compiler_params=pltpu.CompilerParams(use_tc_tiling_on_sc=False),
    )
    def k(m_flat, dsts, bnd, out, bnd_v, dv0, dv1, vv0, vv1, acc,
          semA, semB):
        w = lax.axis_index("s") * 2 + lax.axis_index("c")
        base = w * NPT

        def init(i, carry):
            acc[pl.ds(i * 16, 16)] = jnp.full((16,), NEG, jnp.float32)
            return carry

        lax.fori_loop(0, NPT * H // 16, init, 0)

        pltpu.sync_copy(bnd, bnd_v)
        bv = bnd_v[pl.ds(w, 16)]
        start = bv[0]
        end = bv[1]
        c0 = (start // C) * C
        nch = jnp.maximum((end - c0 + C - 1) // C, 0)

        dv = (dv0, dv1)
        vv = (vv0, vv1)
        sems = (semA, semB)

        def startc(i, b):
            pltpu.async_copy(m_flat.at[pl.ds((c0 + i * C) * H, C * H)],
                             vv[b], sems[b])

        def proc(i, b):
            off = c0 + i * C
            pltpu.sync_copy(dsts.at[pl.ds(off, C)], dv[b])
            pltpu.make_async_copy(m_flat.at[pl.ds((c0 + i * C) * H, C * H)],
                                  vv[b], sems[b]).wait()

            def grp(gg, cc2):
                dvec = dv[b][pl.ds(gg * 16, 16)] - base
                for lane in range(16):
                    e = off + gg * 16 + lane
                    valid = jnp.logical_and(e >= start, e < end)
                    dloc = jnp.where(valid, dvec[lane], 0)
                    abase = dloc * H
                    vbase = (gg * 16 + lane) * H
                    for cc in range(H // 16):
                        v = vv[b][pl.ds(vbase + cc * 16, 16)]
                        v = jnp.where(valid, v,
                                      jnp.full((16,), NEG, jnp.float32))
                        a = acc[pl.ds(abase + cc * 16, 16)]
                        acc[pl.ds(abase + cc * 16, 16)] = jnp.maximum(a, v)
                return cc2

            lax.fori_loop(0, C // 16, grp, 0)

        @pl.when(nch > 0)
        def _():
            startc(0, 0)

        def body(g, carry):
            i0 = 2 * g
            i1 = i0 + 1

            @pl.when(i1 < nch)
            def _():
                startc(i1, 1)

            proc(i0, 0)

            @pl.when(i1 + 1 < nch)
            def _():
                startc(i1 + 1, 0)

            @pl.when(i1 < nch)
            def _():
                proc(i1, 1)

            return carry

        lax.fori_loop(0, (nch + 1) // 2, body, 0)
        pltpu.sync_copy(acc, out.at[pl.ds(base * H, NPT * H)])

    return k


def _dot(a, b):
    return jnp.dot(a, b, preferred_element_type=jnp.float32)


def _full(shape):
    return pl.BlockSpec(shape, lambda *_: (0,) * len(shape))


def _ln(z, g, b, eps=1e-5):
    mu = jnp.mean(z, axis=-1, keepdims=True)
    va = jnp.mean((z - mu) * (z - mu), axis=-1, keepdims=True)
    return (z - mu) / jnp.sqrt(va + eps) * g + b


def _expand_mat():
    hh = lax.broadcasted_iota(jnp.int32, (HEADS, H), 0)
    jj = lax.broadcasted_iota(jnp.int32, (HEADS, H), 1)
    return (jj // CH == hh).astype(jnp.float32)


def _tc_enc(x_p, W, b, g, bb):
    def body(x_ref, W_ref, b_ref, g_ref, bb_ref, o_ref):
        t = _dot(x_ref[...], W_ref[...]) + b_ref[...]
        t = t / jnp.sqrt(jnp.float32(1.0 + 1e-5)) * g_ref[...] + bb_ref[...]
        o_ref[...] = jnp.maximum(t, 0.0)

    return pl.pallas_call(
        body,
        grid=(NPAD // BN,),
        in_specs=[pl.BlockSpec((BN, D_IN), lambda i: (i, 0)),
                  _full((D_IN, H)), _full((1, H)), _full((1, H)), _full((1, H))],
        out_specs=pl.BlockSpec((BN, H), lambda i: (i, 0)),
        out_shape=jax.ShapeDtypeStruct((NPAD, H), jnp.float32),
    )(x_p, W, b.reshape(1, H), g.reshape(1, H), bb.reshape(1, H))


def _tc_eh(ea_p, W, b, g, bb):
    def body(a_ref, W_ref, b_ref, g_ref, bb_ref, o_ref):
        t = jnp.maximum(_dot(a_ref[...], W_ref[...]) + b_ref[...], 0.0)
        o_ref[...] = _ln(t, g_ref[...], bb_ref[...])

    return pl.pallas_call(
        body,
        grid=(EPAD // BE,),
        in_specs=[pl.BlockSpec((BE, 16), lambda i: (i, 0)),
                  _full((16, H)), _full((1, H)), _full((1, H)), _full((1, H))],
        out_specs=pl.BlockSpec((BE, H), lambda i: (i, 0)),
        out_shape=jax.ShapeDtypeStruct((EPAD, H), jnp.float32),
    )(ea_p, W, b.reshape(1, H), g.reshape(1, H), bb.reshape(1, H))


def _dinv_from_ptr(ptr2_blk):
    deg = (ptr2_blk[:, 1:2] - ptr2_blk[:, 0:1] + 1).astype(jnp.float32)
    return lax.rsqrt(deg)


def _tc_gcn_pre(h, ptr2, W):
    def body(h_ref, p_ref, W_ref, o_ref):
        dinv = _dinv_from_ptr(p_ref[...])
        o_ref[...] = _dot(h_ref[...], W_ref[...]) * dinv

    return pl.pallas_call(
        body,
        grid=(NPAD // BN,),
        in_specs=[pl.BlockSpec((BN, H), lambda i: (i, 0)),
                  pl.BlockSpec((BN, 2), lambda i: (i, 0)),
                  _full((H, H))],
        out_specs=pl.BlockSpec((BN, H), lambda i: (i, 0)),
        out_shape=jax.ShapeDtypeStruct((NPAD, H), jnp.float32),
    )(h, ptr2, W)


def _tc_gcn_post(S0, S1, y, ptr2, b, residual):
    def body(s0_ref, s1_ref, y_ref, p_ref, b_ref, o_ref):
        dinv = _dinv_from_ptr(p_ref[...])
        t = dinv * (s0_ref[...] + s1_ref[...] + y_ref[...]) + b_ref[...]
        r = jnp.maximum(t, 0.0)
        o_ref[...] = r + t if residual else r

    return pl.pallas_call(
        body,
        grid=(NPAD // BN,),
        in_specs=[pl.BlockSpec((BN, H), lambda i: (i, 0))] * 3 +
                 [pl.BlockSpec((BN, 2), lambda i: (i, 0)), _full((1, H))],
        out_specs=pl.BlockSpec((BN, H), lambda i: (i, 0)),
        out_shape=jax.ShapeDtypeStruct((NPAD, H), jnp.float32),
    )(S0, S1, y, ptr2, b.reshape(1, H))


def _tc_gat_pre(h, W, Ms, Md):
    def body(h_ref, W_ref, Ms_ref, Md_ref, t_ref, td_ref):
        xw = _dot(h_ref[...], W_ref[...])
        as_ = _dot(xw, Ms_ref[...])
        ad_ = _dot(xw, Md_ref[...])
        t_ref[...] = jnp.concatenate([xw, as_, ad_], axis=1)
        td_ref[...] = jnp.concatenate([ad_, jnp.zeros_like(ad_)], axis=1)

    return pl.pallas_call(
        body,
        grid=(NPAD // BN,),
        in_specs=[pl.BlockSpec((BN, H), lambda i: (i, 0)),
                  _full((H, H)), _full((H, HEADS)), _full((H, HEADS))],
        out_specs=[pl.BlockSpec((BN, H + 2 * HEADS), lambda i: (i, 0)),
                   pl.BlockSpec((BN, 2 * HEADS), lambda i: (i, 0))],
        out_shape=[jax.ShapeDtypeStruct((NPAD, H + 2 * HEADS), jnp.float32),
                   jax.ShapeDtypeStruct((NPAD, 2 * HEADS), jnp.float32)],
    )(h, W, Ms, Md)


def _tc_gat_edge(G, Gd):
    def body(g_ref, gd_ref, o_ref):
        R = _expand_mat()
        g = g_ref[...]
        e = g[:, H:H + HEADS] + gd_ref[...][:, 0:HEADS]
        e = jnp.where(e >= 0, e, 0.2 * e)
        ex = jnp.exp(e)
        num = g[:, 0:H] * _dot(ex, R)
        o_ref[...] = jnp.concatenate(
            [num, ex, jnp.zeros_like(ex)], axis=1)

    return pl.pallas_call(
        body,
        grid=(EPAD // BE,),
        in_specs=[pl.BlockSpec((BE, H + 2 * HEADS), lambda i: (i, 0)),
                  pl.BlockSpec((BE, 2 * HEADS), lambda i: (i, 0))],
        out_specs=pl.BlockSpec((BE, H + 2 * HEADS), lambda i: (i, 0)),
        out_shape=jax.ShapeDtypeStruct((EPAD, H + 2 * HEADS), jnp.float32),
    )(G, Gd)


def _tc_gat_post(S0, S1, T, b, residual):
    def body(s0_ref, s1_ref, t_ref, b_ref, o_ref):
        R = _expand_mat()
        s = s0_ref[...] + s1_ref[...]
        tt = t_ref[...]
        xw = tt[:, 0:H]
        es = tt[:, H:H + HEADS] + tt[:, H + HEADS:H + 2 * HEADS]
        es = jnp.where(es >= 0, es, 0.2 * es)
        exs = jnp.exp(es)
        num = s[:, 0:H] + xw * _dot(exs, R)
        den = _dot(s[:, H:H + HEADS] + exs, R)
        t = num / jnp.maximum(den, 1e-16) + b_ref[...]
        r = jnp.maximum(t, 0.0)
        o_ref[...] = r + t if residual else r

    return pl.pallas_call(
        body,
        grid=(NPAD // BN,),
        in_specs=[pl.BlockSpec((BN, H + 2 * HEADS), lambda i: (i, 0))] * 2 +
                 [pl.BlockSpec((BN, H + 2 * HEADS), lambda i: (i, 0)),
                  _full((1, H))],
        out_specs=pl.BlockSpec((BN, H), lambda i: (i, 0)),
        out_shape=jax.ShapeDtypeStruct((NPAD, H), jnp.float32),
    )(S0, S1, T, b.reshape(1, H))


def _tc_ec_edge(Hs, Hd, W1a, W1b, b1, W2, b2):
    def body(hs_ref, hd_ref, w1a, w1b, b1_ref, w2, b2_ref, o_ref):
        xi = hd_ref[...]
        xj = hs_ref[...]
        t = _dot(xi, w1a[...]) + _dot(xj - xi, w1b[...]) + b1_ref[...]
        t = jnp.maximum(t, 0.0)
        o_ref[...] = _dot(t, w2[...]) + b2_ref[...]

    return pl.pallas_call(
        body,
        grid=(EPAD // BE,),
        in_specs=[pl.BlockSpec((BE, H), lambda i: (i, 0))] * 2 +
                 [_full((H, H)), _full((H, H)), _full((1, H)),
                  _full((H, H)), _full((1, H))],
        out_specs=pl.BlockSpec((BE, H), lambda i: (i, 0)),
        out_shape=jax.ShapeDtypeStruct((EPAD, H), jnp.float32),
    )(Hs, Hd, W1a, W1b, b1.reshape(1, H), W2, b2.reshape(1, H))


def _tc_ec_post(M, residual):
    def body(m_ref, o_ref):
        m = m_ref[...]
        t = jnp.where(jnp.isfinite(m), m, 0.0)
        r = jnp.maximum(t, 0.0)
        o_ref[...] = r + t if residual else r

    return pl.pallas_call(
        body,
        grid=(NPAD // BN,),
        in_specs=[pl.BlockSpec((BN, H), lambda i: (i, 0))],
        out_specs=pl.BlockSpec((BN, H), lambda i: (i, 0)),
        out_shape=jax.ShapeDtypeStruct((NPAD, H), jnp.float32),
    )(M)


def _tc_global(h, gc_W1, gc_b1, gc_W2, gc_b2, ep_W1d, ep_b1):
    def body(h_ref, w1, b1, w2, b2, wd, bd, o_ref):
        rid = lax.broadcasted_iota(jnp.int32, (NPAD, 1), 0)
        hm = jnp.where(rid < N, h_ref[...], 0.0)
        g = jnp.sum(hm, axis=0, keepdims=True) / jnp.float32(N)
        g = jnp.maximum(_dot(g, w1[...]) + b1[...], 0.0)
        g = _dot(g, w2[...]) + b2[...]
        o_ref[...] = _dot(g, wd[...]) + bd[...]

    return pl.pallas_call(
        body,
        in_specs=[_full((NPAD, H)), _full((H, H)), _full((1, H)),
                  _full((H, H)), _full((1, H)), _full((H, 2 * H)),
                  _full((1, 2 * H))],
        out_specs=_full((1, 2 * H)),
        out_shape=jax.ShapeDtypeStruct((1, 2 * H), jnp.float32),
    )(h, gc_W1, gc_b1.reshape(1, H), gc_W2, gc_b2.reshape(1, H),
      ep_W1d, ep_b1.reshape(1, 2 * H))


def _tc_pred(Ps, Pd, eh, cbias, W1a, W1b, W1c, ln1g, ln1b,
             W2, b2, ln2g, ln2b, W3, b3):
    def body(ps_ref, pd_ref, eh_ref, cb, w1a, w1b, w1c, g1, bb1,
             w2, bb2, g2, bb2n, w3, bb3, o_ref):
        z = (_dot(ps_ref[...], w1a[...]) + _dot(pd_ref[...], w1b[...]) +
             _dot(eh_ref[...], w1c[...]) + cb[...])
        z = jnp.maximum(z, 0.0)
        z = _ln(z, g1[...], bb1[...])
        z = jnp.maximum(_dot(z, w2[...]) + bb2[...], 0.0)
        z = _ln(z, g2[...], bb2n[...])
        o_ref[...] = _dot(z, w3[...]) + bb3[...]

    return pl.pallas_call(
        body,
        grid=(EPAD // BE,),
        in_specs=[pl.BlockSpec((BE, H), lambda i: (i, 0))] * 3 +
                 [_full((1, 2 * H)), _full((H, 2 * H)), _full((H, 2 * H)),
                  _full((H, 2 * H)), _full((1, 2 * H)), _full((1, 2 * H)),
                  _full((2 * H, H)), _full((1, H)), _full((1, H)),
                  _full((1, H)), _full((H, 8)), _full((1, 8))],
        out_specs=pl.BlockSpec((BE, 8), lambda i: (i, 0)),
        out_shape=jax.ShapeDtypeStruct((EPAD, 8), jnp.float32),
    )(Ps, Pd, eh, cbias, W1a, W1b, W1c, ln1g.reshape(1, 2 * H),
      ln1b.reshape(1, 2 * H), W2, b2.reshape(1, H), ln2g.reshape(1, H),
      ln2b.reshape(1, H), W3, b3.reshape(1, 8))


def _gather_pair(t1, i1, t2, i2, D1, D2):
    return _sc_gather_pair(D1, D2)(t1, i1, t2, i2)
def _gcn_agg(t, si, di, z): return _sc_gcn_agg()(t, si, di, z)
def _scatter80(v, i, z): return _sc_scatter_add(H + 2 * HEADS)(v, i, z)
def _segmax(m, d, b): return _sc_segmax()(m, d, b)


def kernel(x, edge_index, edge_attr, params):
    p = params
    src = edge_index[0].astype(jnp.int32)
    dst = edge_index[1].astype(jnp.int32)

    key = jnp.sort(dst * 32768 + src)
    dst_s = key >> 15
    src_s = key & 32767
    ptr = jnp.searchsorted(key, jnp.arange(N + 1, dtype=jnp.int32) * 32768
                           ).astype(jnp.int32)
    ptr2 = jnp.zeros((NPAD, 2), jnp.int32)
    ptr2 = ptr2.at[:N, 0].set(ptr[:N]).at[:N, 1].set(ptr[1:])
    bnd = ptr[jnp.minimum(jnp.arange(33, dtype=jnp.int32) * NPT, N)]
    bnd = jnp.pad(bnd, (0, 15), constant_values=E)

    pad_e = EPAD - E
    dst_sp = jnp.pad(dst_s, (0, pad_e), constant_values=NPAD - 1)
    src_sp = jnp.pad(src_s, (0, pad_e), constant_values=0)
    src_p = jnp.pad(src, (0, pad_e), constant_values=0)
    dst_p = jnp.pad(dst, (0, pad_e), constant_values=0)

    x_p = jnp.pad(x, ((0, NPAD - N), (0, 0)))
    ea_p = jnp.pad(edge_attr, ((0, pad_e), (0, 0)))

    z64 = jnp.zeros((NPS, H), jnp.float32)
    z80 = jnp.zeros((NPS, H + 2 * HEADS), jnp.float32)

    eyeh = jnp.eye(HEADS, dtype=jnp.float32)
    Ms0 = {}
    for j in range(2):
        for nm in ('as', 'ad'):
            a = p[f'gat_{nm}_{j}']
            Ms0[(nm, j)] = (a[:, :, None] * eyeh[:, None, :]).reshape(H, HEADS)

    h = _tc_enc(x_p, p['enc_W'], p['enc_b'], p['enc_bn_g'], p['enc_bn_b'])
    eh = _tc_eh(ea_p, p['ee_W'], p['ee_b'], p['ee_ln_g'], p['ee_ln_b'])

    for i in range(6):
        j = i // 3
        residual = (i % 2 == 1)
        if i % 3 == 0:
            y = _tc_gcn_pre(h, ptr2, p[f'gcn_W_{j}'])
            S = _gcn_agg(y, src_sp, dst_sp, z64)
            h = _tc_gcn_post(S[0], S[1], y, ptr2, p[f'gcn_b_{j}'], residual)
        elif i % 3 == 1:
            T, Td = _tc_gat_pre(h, p[f'gat_W_{j}'],
                                Ms0[('as', j)], Ms0[('ad', j)])
            G, Gd = _gather_pair(T, src_sp, Td, dst_sp,
                                 H + 2 * HEADS, 2 * HEADS)
            numex = _tc_gat_edge(G, Gd)
            S = _scatter80(numex, dst_sp, z80)
            h = _tc_gat_post(S[0], S[1], T, p[f'gat_b_{j}'], residual)
        else:
            Hs, Hd = _gather_pair(h, src_sp, h, dst_sp, H, H)
            W1 = p[f'ec_W1_{j}']
            m = _tc_ec_edge(Hs, Hd, W1[:H], W1[H:], p[f'ec_b1_{j}'],
                            p[f'ec_W2_{j}'], p[f'ec_b2_{j}'])
            M = _segmax(m.reshape(EPAD * H), dst_sp, bnd)
            h = _tc_ec_post(M.reshape(NPAD, H), residual)

    ep_W1 = p['ep_W1']
    cbias = _tc_global(h, p['gc_W1'], p['gc_b1'], p['gc_W2'], p['gc_b2'],
                       ep_W1[3 * H:], p['ep_b1'])

    Ps, Pd = _gather_pair(h, src_p, h, dst_p, H, H)
    W3 = jnp.pad(p['ep_W3'], ((0, 0), (0, 7)))
    b3 = jnp.pad(p['ep_b3'], (0, 7))
    logits = _tc_pred(Ps, Pd, eh, cbias,
                      ep_W1[:H], ep_W1[H:2 * H], ep_W1[2 * H:3 * H],
                      p['ep_ln1_g'], p['ep_ln1_b'],
                      p['ep_W2'], p['ep_b2'], p['ep_ln2_g'], p['ep_ln2_b'],
                      W3, b3)
    return logits[:E, 0]
